# baseline (speedup 1.0000x reference)
import jax
import jax.numpy as jnp
from jax import lax
from jax.experimental import pallas as pl
from jax.experimental.pallas import tpu as pltpu
from jax.experimental.pallas import tpu_sc as plsc

_BATCH = 16384
_DIM = 128
_NC = 2
_NS = 16
_NW = _NC * _NS
_NCHUNK = 2
_CHUNK = _BATCH // _NCHUNK
_BPW = _CHUNK // _NW
_P = 128
_NBUF = 4
_LEAD = 3


def _sc_gather_body(u_hbm, ui_hbm, i_hbm, ii_hbm, ou_hbm, oi_hbm,
                    uidx_v, iidx_v, bufs_and_sems):
    bufs = bufs_and_sems[:_NBUF]
    gsem = bufs_and_sems[_NBUF:2 * _NBUF]
    wsem = bufs_and_sems[2 * _NBUF:]
    wid = lax.axis_index("s") * _NC + lax.axis_index("c")
    base = wid * _BPW
    pltpu.sync_copy(ui_hbm.at[pl.ds(base, _BPW)], uidx_v)
    pltpu.sync_copy(ii_hbm.at[pl.ds(base, _BPW)], iidx_v)

    npieces = _BPW // _P
    items = []
    for p in range(npieces):
        items.append((u_hbm, uidx_v, ou_hbm, p))
        items.append((i_hbm, iidx_v, oi_hbm, p))
    n = len(items)

    def start_gather(j):
        tab, idx, _, p = items[j]
        b = j % _NBUF
        return pltpu.async_copy(
            tab.at[idx.at[pl.ds(p * _P, _P)]], bufs[b], gsem[b])

    gcp = {}
    wcp = {}
    for j in range(min(_LEAD, n)):
        gcp[j] = start_gather(j)
    for j in range(n):
        b = j % _NBUF
        gcp[j].wait()
        _, _, out, p = items[j]
        wcp[j] = pltpu.async_copy(
            bufs[b], out.at[pl.ds(base + p * _P, _P)], wsem[b])
        nxt = j + _LEAD
        if nxt < n:
            prev = nxt - _NBUF
            if prev >= 0:
                wcp[prev].wait()
            gcp[nxt] = start_gather(nxt)
    for j in range(max(0, n - _NBUF), n):
        wcp[j].wait()


def _sc_gather(uEmbed, userIdx, iEmbed, itemIdx):
    mesh = plsc.VectorSubcoreMesh(core_axis_name="c", subcore_axis_name="s")
    scratch = (
        [pltpu.VMEM((_BPW,), jnp.int32), pltpu.VMEM((_BPW,), jnp.int32)]
        + [pltpu.VMEM((_P, _DIM), jnp.float32) for _ in range(_NBUF)]
        + [pltpu.SemaphoreType.DMA for _ in range(2 * _NBUF)]
    )

    def body(u_hbm, ui_hbm, i_hbm, ii_hbm, ou_hbm, oi_hbm, uidx_v, iidx_v,
             *bufs_and_sems):
        _sc_gather_body(u_hbm, ui_hbm, i_hbm, ii_hbm, ou_hbm, oi_hbm,
                        uidx_v, iidx_v, bufs_and_sems)

    k = pl.kernel(
        body,
        mesh=mesh,
        out_type=(
            jax.ShapeDtypeStruct((_CHUNK, _DIM), jnp.float32),
            jax.ShapeDtypeStruct((_CHUNK, _DIM), jnp.float32),
        ),
        scratch_types=scratch,
    )
    return k(uEmbed, userIdx, iEmbed, itemIdx)


_HID = 64
_BB = 8192


def _mlp_body(u_ref, i_ref, wu_ref, wi_ref, b1_ref, w2_ref, b2_ref, o_ref):
    u = u_ref[...].astype(jnp.bfloat16)
    i = i_ref[...].astype(jnp.bfloat16)
    h = jnp.dot(u, wu_ref[...], preferred_element_type=jnp.float32)
    h = h + jnp.dot(i, wi_ref[...], preferred_element_type=jnp.float32)
    h = jnp.maximum(h + b1_ref[...], 0.0)
    z = jnp.sum(h * w2_ref[...], axis=1, keepdims=True)
    o_ref[...] = jax.nn.sigmoid(z + b2_ref[...])


def _mlp(uG, iG, wu, wi, b1, w2, b2):
    return pl.pallas_call(
        _mlp_body,
        grid=(_CHUNK // _BB,),
        in_specs=[
            pl.BlockSpec((_BB, _DIM), lambda j: (j, 0)),
            pl.BlockSpec((_BB, _DIM), lambda j: (j, 0)),
            pl.BlockSpec((_DIM, _HID), lambda j: (0, 0)),
            pl.BlockSpec((_DIM, _HID), lambda j: (0, 0)),
            pl.BlockSpec((1, _HID), lambda j: (0, 0)),
            pl.BlockSpec((1, _HID), lambda j: (0, 0)),
            pl.BlockSpec((1, 1), lambda j: (0, 0)),
        ],
        out_specs=pl.BlockSpec((_BB, 1), lambda j: (j, 0)),
        out_shape=jax.ShapeDtypeStruct((_CHUNK, 1), jnp.float32),
    )(uG, iG, wu, wi, b1, w2, b2)


def kernel(userIdx, itemIdx, uEmbed, iEmbed, W_cvr, b_cvr, W_cvr1, b_cvr1):
    userIdx = userIdx.astype(jnp.int32)
    itemIdx = itemIdx.astype(jnp.int32)
    wu = W_cvr[:, :_DIM].T.astype(jnp.bfloat16)
    wi = W_cvr[:, _DIM:].T.astype(jnp.bfloat16)
    b1 = b_cvr.reshape(1, _HID)
    w2 = W_cvr1
    b2 = b_cvr1.reshape(1, 1)
    gathered = []
    for c in range(_NCHUNK):
        sl = slice(c * _CHUNK, (c + 1) * _CHUNK)
        gathered.append(_sc_gather(uEmbed, userIdx[sl], iEmbed, itemIdx[sl]))
    outs = [_mlp(uG, iG, wu, wi, b1, w2, b2) for uG, iG in gathered]
    return jnp.concatenate(outs, axis=0).reshape(-1)


# speedup vs baseline: 1.0291x; 1.0291x over previous
import jax
import jax.numpy as jnp
from jax import lax
from jax.experimental import pallas as pl
from jax.experimental.pallas import tpu as pltpu
from jax.experimental.pallas import tpu_sc as plsc

_BATCH = 16384
_DIM = 128
_NC = 2
_NS = 16
_NW = _NC * _NS
_BPW = _BATCH // _NW
_P = 128
_NBUF = 4
_LEAD = 3


def _sc_gather_body(u_hbm, ui_hbm, i_hbm, ii_hbm, ou_hbm, oi_hbm,
                    uidx_v, iidx_v, bufs_and_sems):
    bufs = bufs_and_sems[:_NBUF]
    gsem = bufs_and_sems[_NBUF:2 * _NBUF]
    wsem = bufs_and_sems[2 * _NBUF:]
    wid = lax.axis_index("s") * _NC + lax.axis_index("c")
    base = wid * _BPW
    pltpu.sync_copy(ui_hbm.at[pl.ds(base, _BPW)], uidx_v)
    pltpu.sync_copy(ii_hbm.at[pl.ds(base, _BPW)], iidx_v)

    npieces = _BPW // _P
    items = []
    for p in range(npieces):
        items.append((u_hbm, uidx_v, ou_hbm, p))
        items.append((i_hbm, iidx_v, oi_hbm, p))
    n = len(items)

    def start_gather(j):
        tab, idx, _, p = items[j]
        b = j % _NBUF
        return pltpu.async_copy(
            tab.at[idx.at[pl.ds(p * _P, _P)]], bufs[b], gsem[b])

    gcp = {}
    wcp = {}
    for j in range(min(_LEAD, n)):
        gcp[j] = start_gather(j)
    for j in range(n):
        b = j % _NBUF
        gcp[j].wait()
        _, _, out, p = items[j]
        wcp[j] = pltpu.async_copy(
            bufs[b], out.at[pl.ds(base + p * _P, _P)], wsem[b])
        nxt = j + _LEAD
        if nxt < n:
            prev = nxt - _NBUF
            if prev >= 0:
                wcp[prev].wait()
            gcp[nxt] = start_gather(nxt)
    for j in range(max(0, n - _NBUF), n):
        wcp[j].wait()


def _sc_gather(uEmbed, userIdx, iEmbed, itemIdx):
    mesh = plsc.VectorSubcoreMesh(core_axis_name="c", subcore_axis_name="s")
    scratch = (
        [pltpu.VMEM((_BPW,), jnp.int32), pltpu.VMEM((_BPW,), jnp.int32)]
        + [pltpu.VMEM((_P, _DIM), jnp.float32) for _ in range(_NBUF)]
        + [pltpu.SemaphoreType.DMA for _ in range(2 * _NBUF)]
    )

    def body(u_hbm, ui_hbm, i_hbm, ii_hbm, ou_hbm, oi_hbm, uidx_v, iidx_v,
             *bufs_and_sems):
        _sc_gather_body(u_hbm, ui_hbm, i_hbm, ii_hbm, ou_hbm, oi_hbm,
                        uidx_v, iidx_v, bufs_and_sems)

    k = pl.kernel(
        body,
        mesh=mesh,
        out_type=(
            jax.ShapeDtypeStruct((_BATCH, _DIM), jnp.float32),
            jax.ShapeDtypeStruct((_BATCH, _DIM), jnp.float32),
        ),
        scratch_types=scratch,
    )
    return k(uEmbed, userIdx, iEmbed, itemIdx)


_HID = 64
_BB = 8192


_PB = 2048
_TNBUF = 3
_TLEAD = 2


def _mlp_body(u_hbm, i_hbm, wu_ref, wi_ref, b1_ref, w2_ref, b2_ref, o_ref,
              *scr):
    ubufs = scr[:_TNBUF]
    ibufs = scr[_TNBUF:2 * _TNBUF]
    usems = scr[2 * _TNBUF:3 * _TNBUF]
    isems = scr[3 * _TNBUF:]
    n = _BATCH // _PB

    def start(j):
        b = j % _TNBUF
        uc = pltpu.make_async_copy(
            u_hbm.at[pl.ds(j * _PB, _PB), :], ubufs[b], usems[b])
        ic = pltpu.make_async_copy(
            i_hbm.at[pl.ds(j * _PB, _PB), :], ibufs[b], isems[b])
        uc.start()
        ic.start()
        return uc, ic

    cps = {}
    for j in range(min(_TLEAD, n)):
        cps[j] = start(j)
    for j in range(n):
        b = j % _TNBUF
        nxt = j + _TLEAD
        if nxt < n:
            cps[nxt] = start(nxt)
        uc, ic = cps[j]
        uc.wait()
        ic.wait()
        u = ubufs[b][...].astype(jnp.bfloat16)
        i = ibufs[b][...].astype(jnp.bfloat16)
        h = jnp.dot(u, wu_ref[...], preferred_element_type=jnp.float32)
        h = h + jnp.dot(i, wi_ref[...], preferred_element_type=jnp.float32)
        h = jnp.maximum(h + b1_ref[...], 0.0)
        z = jnp.sum(h * w2_ref[...], axis=1, keepdims=True)
        o_ref[pl.ds(j * _PB, _PB), :] = jax.nn.sigmoid(z + b2_ref[...])


def _mlp(uG, iG, wu, wi, b1, w2, b2):
    scratch = (
        [pltpu.VMEM((_PB, _DIM), jnp.float32) for _ in range(2 * _TNBUF)]
        + [pltpu.SemaphoreType.DMA for _ in range(2 * _TNBUF)]
    )
    return pl.pallas_call(
        _mlp_body,
        grid=(1,),
        in_specs=[
            pl.BlockSpec(memory_space=pltpu.MemorySpace.HBM),
            pl.BlockSpec(memory_space=pltpu.MemorySpace.HBM),
            pl.BlockSpec((_DIM, _HID), lambda j: (0, 0)),
            pl.BlockSpec((_DIM, _HID), lambda j: (0, 0)),
            pl.BlockSpec((1, _HID), lambda j: (0, 0)),
            pl.BlockSpec((1, _HID), lambda j: (0, 0)),
            pl.BlockSpec((1, 1), lambda j: (0, 0)),
        ],
        out_specs=pl.BlockSpec((_BATCH, 1), lambda j: (0, 0)),
        out_shape=jax.ShapeDtypeStruct((_BATCH, 1), jnp.float32),
        scratch_shapes=scratch,
    )(uG, iG, wu, wi, b1, w2, b2)


def kernel(userIdx, itemIdx, uEmbed, iEmbed, W_cvr, b_cvr, W_cvr1, b_cvr1):
    userIdx = userIdx.astype(jnp.int32)
    itemIdx = itemIdx.astype(jnp.int32)
    uG, iG = _sc_gather(uEmbed, userIdx, iEmbed, itemIdx)
    wu = W_cvr[:, :_DIM].T.astype(jnp.bfloat16)
    wi = W_cvr[:, _DIM:].T.astype(jnp.bfloat16)
    b1 = b_cvr.reshape(1, _HID)
    w2 = W_cvr1
    b2 = b_cvr1.reshape(1, 1)
    out = _mlp(uG, iG, wu, wi, b1, w2, b2)
    return out.reshape(-1)


# speedup vs baseline: 1.1049x; 1.0737x over previous
import jax
import jax.numpy as jnp
from jax import lax
from jax.experimental import pallas as pl
from jax.experimental.pallas import tpu as pltpu
from jax.experimental.pallas import tpu_sc as plsc

_BATCH = 16384
_DIM = 128
_NC = 2
_NS = 16
_NW = _NC * _NS
_BPW = _BATCH // _NW
_P = 128
_NBUF = 4
_LEAD = 3


def _sc_gather_body(u_hbm, ui_hbm, i_hbm, ii_hbm, ou_hbm, oi_hbm,
                    uidx_v, iidx_v, bufs_and_sems):
    bufs = bufs_and_sems[:_NBUF]
    gsem = bufs_and_sems[_NBUF:2 * _NBUF]
    wsem = bufs_and_sems[2 * _NBUF:]
    wid = lax.axis_index("s") * _NC + lax.axis_index("c")
    base = wid * _BPW
    uic = pltpu.async_copy(ui_hbm.at[pl.ds(base, _BPW)], uidx_v, gsem[0])
    iic = pltpu.async_copy(ii_hbm.at[pl.ds(base, _BPW)], iidx_v, gsem[1])
    uic.wait()
    iic.wait()

    npieces = _BPW // _P
    items = []
    for p in range(npieces):
        items.append((u_hbm, uidx_v, ou_hbm, p))
        items.append((i_hbm, iidx_v, oi_hbm, p))
    n = len(items)

    def start_gather(j):
        tab, idx, _, p = items[j]
        b = j % _NBUF
        return pltpu.async_copy(
            tab.at[idx.at[pl.ds(p * _P, _P)]], bufs[b], gsem[b])

    gcp = {}
    wcp = {}
    for j in range(min(_LEAD, n)):
        gcp[j] = start_gather(j)
    for j in range(n):
        b = j % _NBUF
        gcp[j].wait()
        _, _, out, p = items[j]
        wcp[j] = pltpu.async_copy(
            bufs[b], out.at[pl.ds(base + p * _P, _P)], wsem[b])
        nxt = j + _LEAD
        if nxt < n:
            prev = nxt - _NBUF
            if prev >= 0:
                wcp[prev].wait()
            gcp[nxt] = start_gather(nxt)
    for j in range(max(0, n - _NBUF), n):
        wcp[j].wait()


def _sc_gather(uEmbed, userIdx, iEmbed, itemIdx):
    mesh = plsc.VectorSubcoreMesh(core_axis_name="c", subcore_axis_name="s")
    scratch = (
        [pltpu.VMEM((_BPW,), jnp.int32), pltpu.VMEM((_BPW,), jnp.int32)]
        + [pltpu.VMEM((_P, _DIM), jnp.float32) for _ in range(_NBUF)]
        + [pltpu.SemaphoreType.DMA for _ in range(2 * _NBUF)]
    )

    def body(u_hbm, ui_hbm, i_hbm, ii_hbm, ou_hbm, oi_hbm, uidx_v, iidx_v,
             *bufs_and_sems):
        _sc_gather_body(u_hbm, ui_hbm, i_hbm, ii_hbm, ou_hbm, oi_hbm,
                        uidx_v, iidx_v, bufs_and_sems)

    k = pl.kernel(
        body,
        mesh=mesh,
        out_type=(
            jax.ShapeDtypeStruct((_BATCH, _DIM), jnp.float32),
            jax.ShapeDtypeStruct((_BATCH, _DIM), jnp.float32),
        ),
        scratch_types=scratch,
    )
    return k(uEmbed, userIdx, iEmbed, itemIdx)


_HID = 64
_BB = 8192


def _mlp_body(u_ref, i_ref, w_ref, b1_ref, w2_ref, b2_ref, o_ref):
    u = u_ref[...].astype(jnp.bfloat16)
    i = i_ref[...].astype(jnp.bfloat16)
    w = w_ref[...].astype(jnp.bfloat16)
    dn = (((1,), (1,)), ((), ()))
    h = jax.lax.dot_general(u, w[:, :_DIM], dn,
                            preferred_element_type=jnp.float32)
    h = h + jax.lax.dot_general(i, w[:, _DIM:], dn,
                                preferred_element_type=jnp.float32)
    h = jnp.maximum(h + b1_ref[...], 0.0)
    z = jnp.sum(h * w2_ref[...], axis=1)
    o_ref[...] = jax.nn.sigmoid(z + b2_ref[...])


def _mlp(uG, iG, W_cvr, b_cvr, W_cvr1, b_cvr1):
    return pl.pallas_call(
        _mlp_body,
        grid=(_BATCH // _BB,),
        in_specs=[
            pl.BlockSpec((_BB, _DIM), lambda j: (j, 0)),
            pl.BlockSpec((_BB, _DIM), lambda j: (j, 0)),
            pl.BlockSpec((_HID, 2 * _DIM), lambda j: (0, 0)),
            pl.BlockSpec((1, _HID), lambda j: (0, 0)),
            pl.BlockSpec((1, _HID), lambda j: (0, 0)),
            pl.BlockSpec((1,), lambda j: (0,)),
        ],
        out_specs=pl.BlockSpec((_BB,), lambda j: (j,)),
        out_shape=jax.ShapeDtypeStruct((_BATCH,), jnp.float32),
    )(uG, iG, W_cvr, b_cvr.reshape(1, _HID), W_cvr1, b_cvr1)


def kernel(userIdx, itemIdx, uEmbed, iEmbed, W_cvr, b_cvr, W_cvr1, b_cvr1):
    userIdx = userIdx.astype(jnp.int32)
    itemIdx = itemIdx.astype(jnp.int32)
    uG, iG = _sc_gather(uEmbed, userIdx, iEmbed, itemIdx)
    return _mlp(uG, iG, W_cvr, b_cvr, W_cvr1, b_cvr1)
